# Initial kernel scaffold; baseline (speedup 1.0000x reference)
import functools

import jax
import jax.numpy as jnp
from jax import lax
from jax.experimental import pallas as pl
from jax.experimental.pallas import tpu as pltpu
from jax.experimental.pallas import tpu_sc as plsc

_B, _H, _W, _C = 16, 16, 16, 768
_N = 4096
_ROWS = _B * _N
_NW = 32
_RPW = _ROWS // _NW
_CH = 64
_NCH = _RPW // _CH

_mesh = plsc.VectorSubcoreMesh(core_axis_name="c", subcore_axis_name="s")


@functools.partial(
    pl.kernel,
    mesh=_mesh,
    out_type=jax.ShapeDtypeStruct((_ROWS, _C), jnp.float32),
    scratch_types=[
        pltpu.VMEM((2 * _RPW,), jnp.int32),
        pltpu.VMEM((_RPW,), jnp.int32),
        pltpu.VMEM((_CH, _C), jnp.float32),
        pltpu.SemaphoreType.DMA,
    ],
)
def _sc_gather(table_hbm, idx_hbm, out_hbm, raw_v, rid_v, buf, gsem):
    wid = lax.axis_index("s") * 2 + lax.axis_index("c")
    base_row = wid * _RPW
    bval = (wid // 2) * (_H * _W)

    pltpu.sync_copy(idx_hbm.at[pl.ds(base_row * 2, 2 * _RPW)], raw_v)

    lane = jnp.arange(16, dtype=jnp.int32)

    def transform(k, carry):
        pos = k * 32 + lane * 2
        i = plsc.load_gather(raw_v, [pos])
        j = plsc.load_gather(raw_v, [pos + 1])
        rid_v[pl.ds(k * 16, 16)] = (i >> 5) * _W + (j >> 5) + bval
        return carry

    lax.fori_loop(0, _RPW // 16, transform, 0)

    def chunk(c, carry):
        pltpu.async_copy(
            table_hbm.at[rid_v.at[pl.ds(c * _CH, _CH)]], buf, gsem
        ).wait()
        pltpu.sync_copy(buf, out_hbm.at[pl.ds(base_row + c * _CH, _CH)])
        return carry

    lax.fori_loop(0, _NCH, chunk, 0)


def kernel(fmap, idx):
    table = fmap.reshape(_B * _H * _W, _C)
    flat_idx = idx.astype(jnp.int32).reshape(-1)
    out = _sc_gather(table, flat_idx)
    return out.reshape(_B, _N, _C)


# speedup vs baseline: 6.6784x; 6.6784x over previous
import functools

import jax
import jax.numpy as jnp
from jax import lax
from jax.experimental import pallas as pl
from jax.experimental.pallas import tpu as pltpu
from jax.experimental.pallas import tpu_sc as plsc

_B, _H, _W, _C = 16, 16, 16, 768
_N = 4096
_ROWS = _B * _N
_NW = 32
_RPW = _ROWS // _NW
_CH = 64
_NCH = _RPW // _CH

_mesh = plsc.VectorSubcoreMesh(core_axis_name="c", subcore_axis_name="s")


@functools.partial(
    pl.kernel,
    mesh=_mesh,
    out_type=jax.ShapeDtypeStruct((_ROWS, _C), jnp.float32),
    compiler_params=pltpu.CompilerParams(needs_layout_passes=False),
    scratch_types=[
        pltpu.VMEM((2 * _RPW,), jnp.int32),
        pltpu.VMEM((_RPW,), jnp.int32),
        pltpu.VMEM((_CH, _C), jnp.float32),
        pltpu.SemaphoreType.DMA,
    ],
)
def _sc_gather(table_hbm, idx_hbm, out_hbm, raw_v, rid_v, buf, gsem):
    wid = lax.axis_index("s") * 2 + lax.axis_index("c")
    base_row = wid * _RPW
    bval = (wid // 2) * (_H * _W)

    pltpu.sync_copy(idx_hbm.at[pl.ds(base_row * 2, 2 * _RPW)], raw_v)

    lane = jnp.arange(16, dtype=jnp.int32)

    def transform(k, carry):
        pos = k * 32 + lane * 2
        i = plsc.load_gather(raw_v, [pos])
        j = plsc.load_gather(raw_v, [pos + 1])
        rid_v[pl.ds(k * 16, 16)] = (i >> 5) * _W + (j >> 5) + bval
        return carry

    lax.fori_loop(0, _RPW // 16, transform, 0)

    def chunk(c, carry):
        pltpu.async_copy(
            table_hbm.at[rid_v.at[pl.ds(c * _CH, _CH)]], buf, gsem
        ).wait()
        pltpu.sync_copy(buf, out_hbm.at[pl.ds(base_row + c * _CH, _CH)])
        return carry

    lax.fori_loop(0, _NCH, chunk, 0)


def kernel(fmap, idx):
    table = fmap.reshape(_B * _H * _W, _C)
    flat_idx = idx.astype(jnp.int32).reshape(-1)
    out = _sc_gather(table, flat_idx)
    return out.reshape(_B, _N, _C)


# speedup vs baseline: 7.2764x; 1.0895x over previous
import functools

import jax
import jax.numpy as jnp
from jax import lax
from jax.experimental import pallas as pl
from jax.experimental.pallas import tpu as pltpu
from jax.experimental.pallas import tpu_sc as plsc

_B, _H, _W, _C = 16, 16, 16, 768
_N = 4096
_ROWS = _B * _N
_NW = 32
_RPW = _ROWS // _NW
_CH = 64
_NCH = _RPW // _CH

_mesh = plsc.VectorSubcoreMesh(core_axis_name="c", subcore_axis_name="s")


@functools.partial(
    pl.kernel,
    mesh=_mesh,
    out_type=jax.ShapeDtypeStruct((_ROWS, _C), jnp.float32),
    compiler_params=pltpu.CompilerParams(needs_layout_passes=False),
    scratch_types=[
        pltpu.VMEM((2 * _RPW,), jnp.int32),
        pltpu.VMEM((_RPW,), jnp.int32),
        pltpu.VMEM((_CH, _C), jnp.float32),
        pltpu.VMEM((_CH, _C), jnp.float32),
        pltpu.SemaphoreType.DMA,
        pltpu.SemaphoreType.DMA,
        pltpu.SemaphoreType.DMA,
        pltpu.SemaphoreType.DMA,
    ],
)
def _sc_gather(table_hbm, idx_hbm, out_hbm, raw_v, rid_v, buf0, buf1,
               gsem0, gsem1, ssem0, ssem1):
    wid = lax.axis_index("s") * 2 + lax.axis_index("c")
    base_row = wid * _RPW
    bval = (wid // 2) * (_H * _W)

    pltpu.sync_copy(idx_hbm.at[pl.ds(base_row * 2, 2 * _RPW)], raw_v)

    lane = jnp.arange(16, dtype=jnp.int32)

    def transform(k, carry):
        pos = k * 32 + lane * 2
        i = plsc.load_gather(raw_v, [pos])
        j = plsc.load_gather(raw_v, [pos + 1])
        rid_v[pl.ds(k * 16, 16)] = (i >> 5) * _W + (j >> 5) + bval
        return carry

    lax.fori_loop(0, _RPW // 16, transform, 0)

    bufs = (buf0, buf1)
    gsems = (gsem0, gsem1)
    ssems = (ssem0, ssem1)

    def start_gather(c):
        return pltpu.async_copy(
            table_hbm.at[rid_v.at[pl.ds(c * _CH, _CH)]], bufs[c % 2],
            gsems[c % 2],
        )

    def start_store(c):
        return pltpu.async_copy(
            bufs[c % 2], out_hbm.at[pl.ds(base_row + c * _CH, _CH)],
            ssems[c % 2],
        )

    g = [None, None]
    s = [None, None]
    for c in range(_NCH):
        p = c % 2
        if s[p] is not None:
            s[p].wait()
        g[p] = start_gather(c)
        if c >= 1:
            q = (c - 1) % 2
            g[q].wait()
            s[q] = start_store(c - 1)
    q = (_NCH - 1) % 2
    g[q].wait()
    s[q] = start_store(_NCH - 1)
    s[0].wait()
    s[1].wait()


def kernel(fmap, idx):
    table = fmap.reshape(_B * _H * _W, _C)
    flat_idx = idx.astype(jnp.int32).reshape(-1)
    out = _sc_gather(table, flat_idx)
    return out.reshape(_B, _N, _C)


# speedup vs baseline: 7.3004x; 1.0033x over previous
import functools

import jax
import jax.numpy as jnp
from jax import lax
from jax.experimental import pallas as pl
from jax.experimental.pallas import tpu as pltpu
from jax.experimental.pallas import tpu_sc as plsc

_B, _H, _W, _C = 16, 16, 16, 768
_N = 4096
_ROWS = _B * _N
_NW = 32
_RPW = _ROWS // _NW
_CH = 64
_NCH = _RPW // _CH

_mesh = plsc.VectorSubcoreMesh(core_axis_name="c", subcore_axis_name="s")


@functools.partial(
    pl.kernel,
    mesh=_mesh,
    out_type=jax.ShapeDtypeStruct((_ROWS, _C), jnp.float32),
    compiler_params=pltpu.CompilerParams(
        needs_layout_passes=False,
        skip_device_barrier=True,
        disable_bounds_checks=True,
        disable_semaphore_checks=True,
    ),
    scratch_types=[
        pltpu.VMEM((2 * _RPW,), jnp.int32),
        pltpu.VMEM((_RPW,), jnp.int32),
        pltpu.VMEM((_CH, _C), jnp.float32),
        pltpu.VMEM((_CH, _C), jnp.float32),
        pltpu.SemaphoreType.DMA,
        pltpu.SemaphoreType.DMA,
        pltpu.SemaphoreType.DMA,
        pltpu.SemaphoreType.DMA,
    ],
)
def _sc_gather(fmap_hbm, idx_hbm, out_hbm, raw_v, rid_v, buf0, buf1,
               gsem0, gsem1, ssem0, ssem1):
    table_hbm = fmap_hbm.reshape(_B * _H * _W, _C)
    wid = lax.axis_index("s") * 2 + lax.axis_index("c")
    base_row = wid * _RPW
    bval = (wid // 2) * (_H * _W)

    pltpu.sync_copy(idx_hbm.at[pl.ds(base_row * 2, 2 * _RPW)], raw_v)

    lane = jnp.arange(16, dtype=jnp.int32)

    def transform(k, carry):
        pos = k * 32 + lane * 2
        i = plsc.load_gather(raw_v, [pos])
        j = plsc.load_gather(raw_v, [pos + 1])
        rid_v[pl.ds(k * 16, 16)] = (i >> 5) * _W + (j >> 5) + bval
        return carry

    lax.fori_loop(0, _RPW // 16, transform, 0)

    bufs = (buf0, buf1)
    gsems = (gsem0, gsem1)
    ssems = (ssem0, ssem1)

    def start_gather(c):
        return pltpu.async_copy(
            table_hbm.at[rid_v.at[pl.ds(c * _CH, _CH)]], bufs[c % 2],
            gsems[c % 2],
        )

    def start_store(c):
        return pltpu.async_copy(
            bufs[c % 2], out_hbm.at[pl.ds(base_row + c * _CH, _CH)],
            ssems[c % 2],
        )

    g = [None, None]
    s = [None, None]
    for c in range(_NCH):
        p = c % 2
        if s[p] is not None:
            s[p].wait()
        g[p] = start_gather(c)
        if c >= 1:
            q = (c - 1) % 2
            g[q].wait()
            s[q] = start_store(c - 1)
    q = (_NCH - 1) % 2
    g[q].wait()
    s[q] = start_store(_NCH - 1)
    s[0].wait()
    s[1].wait()


def kernel(fmap, idx):
    flat_idx = idx.astype(jnp.int32).reshape(-1)
    out = _sc_gather(fmap, flat_idx)
    return out.reshape(_B, _N, _C)
